# baseline (speedup 1.0000x reference)
import functools

import jax
import jax.numpy as jnp
from jax import lax
from jax.experimental import pallas as pl
from jax.experimental.pallas import tpu as pltpu
from jax.experimental.pallas import tpu_sc as plsc

NB, NA, NG, NC = 16, 3, 52, 80
NGTS = 1920
PLANE = NG * NG
NCELL = NB * NA * PLANE
NCH = NC + 5
HALF = PLANE // 2
HBUF = 1360
NW = 32
L = 16
GT_VECS = NGTS // L
W_VECS = HBUF // L

A_W = (0.28, 0.38, 0.9)
A_H = (0.22, 0.48, 0.78)
LN2 = 0.6931471805599453


def _ln(v):
    bits = lax.bitcast_convert_type(v, jnp.int32)
    e = ((bits >> 23) & 0xFF) - 127
    m = lax.bitcast_convert_type((bits & 0x007FFFFF) | 0x3F800000, jnp.float32)
    big = m > 1.4142135623730951
    m = jnp.where(big, m * 0.5, m)
    e = e + jnp.where(big, 1, 0)
    t = (m - 1.0) / (m + 1.0)
    t2 = t * t
    p = jnp.float32(2.0 / 9.0)
    p = 2.0 / 7.0 + t2 * p
    p = 2.0 / 5.0 + t2 * p
    p = 2.0 / 3.0 + t2 * p
    p = 2.0 + t2 * p
    return e.astype(jnp.float32) * LN2 + t * p


def _log_clamped(v):
    safe = jnp.maximum(v, jnp.float32(1e-37))
    return jnp.where(v < 1e-37, -100.0, jnp.maximum(_ln(safe), -100.0))


def _sigmoid(x):
    return 1.0 / (1.0 + jnp.exp(-x))


def _yolo_sc_body(out_hbm, gts_hbm, part_hbm,
                  gts_v, cell_v,
                  win0, win1, win2, ign0, ign1, ign2, conf0, conf1, conf2,
                  wlist_v, idx_v, chan_v, acc_v, sem0, sem1):
    win_r = (win0, win1, win2)
    ign_r = (ign0, ign1, ign2)
    conf_r = (conf0, conf1, conf2)
    cid = lax.axis_index("c")
    sid = lax.axis_index("s")
    wid = sid * 2 + cid
    iota = lax.iota(jnp.int32, L)
    fzero = jnp.zeros((L,), jnp.float32)

    pltpu.sync_copy(gts_hbm, gts_v)

    dmabases = []
    los = []
    for w in range(3):
        hp = wid * 3 + w
        pi = hp // 2
        h = hp - 2 * pi
        b = pi // 3
        a = pi - 3 * b
        src_off = (b * (NA * NCH) + a * NCH + 4) * PLANE + h * (HALF - 8)
        los.append(hp * HALF)
        dmabases.append(pi * PLANE + h * (HALF - 8))
        pltpu.async_copy(out_hbm.at[pl.ds(src_off, HBUF)], conf_r[w], sem0)

    def _init(i, _):
        sl = pl.ds(i * L, L)
        for w in range(3):
            win_r[w][sl] = jnp.full((L,), -1, jnp.int32)
            ign_r[w][sl] = jnp.full((L,), 1.0, jnp.float32)
        return 0
    lax.fori_loop(0, W_VECS, _init, 0)

    for k in range(6):
        acc_v[pl.ds(k * L, L)] = fzero

    lo_v = [jnp.full((L,), los[w], jnp.int32) for w in range(3)]
    hi_v = [jnp.full((L,), los[w] + HALF, jnp.int32) for w in range(3)]
    db_v = [jnp.full((L,), dmabases[w], jnp.int32) for w in range(3)]

    def _gt_pass(i, _):
        g = i * L + iota
        b = gts_v[pl.ds(i * L, L)].astype(jnp.int32)
        gx = gts_v[pl.ds(2 * NGTS + i * L, L)]
        gy = gts_v[pl.ds(3 * NGTS + i * L, L)]
        gw = gts_v[pl.ds(4 * NGTS + i * L, L)]
        gh = gts_v[pl.ds(5 * NGTS + i * L, L)]
        ious = []
        for a in range(NA):
            inter = jnp.minimum(gw, A_W[a]) * jnp.minimum(gh, A_H[a])
            union = gw * gh + A_W[a] * A_H[a] - inter
            ious.append(inter / union)
        best = jnp.zeros((L,), jnp.int32)
        bi = ious[0]
        best = jnp.where(ious[1] > bi, 1, best)
        bi = jnp.maximum(bi, ious[1])
        best = jnp.where(ious[2] > bi, 2, best)
        gi = (NG * gx).astype(jnp.int32)
        gj = (NG * gy).astype(jnp.int32)
        pos = gj * NG + gi
        cell = (b * NA + best) * PLANE + pos
        cell_v[pl.ds(i * L, L)] = cell
        for w in range(3):
            m = (cell >= lo_v[w]) & (cell < hi_v[w])
            lidx = jnp.clip(cell - db_v[w], 0, HBUF - 1)
            plsc.store_scatter(win_r[w], [lidx], g, mask=m)
            for a in range(NA):
                ca = (b * NA + a) * PLANE + pos
                ma = (ca >= lo_v[w]) & (ca < hi_v[w]) & (ious[a] > 0.5)
                la = jnp.clip(ca - db_v[w], 0, HBUF - 1)
                plsc.store_scatter(ign_r[w], [la], fzero, mask=ma)
        return 0
    lax.fori_loop(0, GT_VECS, _gt_pass, 0)

    for w in range(3):
        pltpu.make_async_copy(
            out_hbm.at[pl.ds(0, HBUF)], conf_r[w], sem0).wait()

    for w in range(3):
        def _dense(i, _, w=w):
            sl = pl.ds(i * L, L)
            c = db_v[w] + i * L + iota
            inw = (c >= lo_v[w]) & (c < hi_v[w])
            win = win_r[w][sl]
            ign = ign_r[w][sl]
            x = conf_r[w][sl]
            obj = inw & (win >= 0)
            objf = jnp.where(obj, 1.0, 0.0)
            inwf = jnp.where(inw, 1.0, 0.0)
            noobjf = (inwf - objf) * ign
            p = _sigmoid(x)
            val = jnp.where(obj, p, 1.0 - p)
            bce = -_log_clamped(val)
            acc_v[pl.ds(0, L)] = acc_v[pl.ds(0, L)] + objf
            acc_v[pl.ds(L, L)] = acc_v[pl.ds(L, L)] + noobjf
            acc_v[pl.ds(2 * L, L)] = acc_v[pl.ds(2 * L, L)] + objf * bce
            acc_v[pl.ds(3 * L, L)] = acc_v[pl.ds(3 * L, L)] + noobjf * bce
            return 0
        lax.fori_loop(0, W_VECS, _dense, 0)

    def _detect(i, cnt):
        g = i * L + iota
        cell = cell_v[pl.ds(i * L, L)]
        iswin = jnp.zeros((L,), jnp.bool_)
        for w in range(3):
            m = (cell >= lo_v[w]) & (cell < hi_v[w])
            lidx = jnp.clip(cell - db_v[w], 0, HBUF - 1)
            stored = plsc.load_gather(win_r[w], [lidx])
            iswin = iswin | (m & (stored == g))
        plsc.store_compressed(wlist_v.at[pl.ds(cnt, L)], g, mask=iswin)
        return cnt + jnp.sum(iswin.astype(jnp.int32))
    nwin = lax.fori_loop(0, GT_VECS, _detect, jnp.int32(0))

    chan_id = [j * L + iota for j in range(6)]

    def _winner(widx, _):
        wsplat = jnp.full((L,), widx, jnp.int32)
        g = plsc.load_gather(wlist_v, [wsplat])
        cell = plsc.load_gather(cell_v, [g])
        pi = cell // PLANE
        pos = cell - pi * PLANE
        b = pi // NA
        a = pi - NA * b
        base = (b * (NA * NCH) + a * NCH) * PLANE + pos
        for j in range(6):
            ch = jnp.minimum(chan_id[j], NCH - 1)
            idx_v[pl.ds(j * L, L)] = base + ch * PLANE
        pltpu.async_copy(out_hbm.at[idx_v], chan_v, sem1)

        gxs = plsc.load_gather(gts_v, [2 * NGTS + g])
        gys = plsc.load_gather(gts_v, [3 * NGTS + g])
        gws = plsc.load_gather(gts_v, [4 * NGTS + g])
        ghs = plsc.load_gather(gts_v, [5 * NGTS + g])
        lab = plsc.load_gather(gts_v, [NGTS + g]).astype(jnp.int32)
        aw = jnp.where(a == 0, A_W[0], jnp.where(a == 1, A_W[1], A_W[2]))
        ah = jnp.where(a == 0, A_H[0], jnp.where(a == 1, A_H[1], A_H[2]))
        txf = NG * gxs
        tyf = NG * gys
        txs = txf - txf.astype(jnp.int32).astype(jnp.float32)
        tys = tyf - tyf.astype(jnp.int32).astype(jnp.float32)
        tw = _ln(gws / aw)
        th = _ln(ghs / ah)
        pltpu.make_async_copy(out_hbm.at[idx_v], chan_v, sem1).wait()

        reg = fzero
        cls = fzero
        for j in range(6):
            x = chan_v[pl.ds(j * L, L)]
            ch = chan_id[j]
            p = _sigmoid(x)
            tgt = jnp.where(ch == 0, txs, jnp.where(ch == 1, tys,
                  jnp.where(ch == 2, tw, th)))
            v = jnp.where(ch < 2, p, x)
            d = v - tgt
            reg = reg + jnp.where(ch < 4, d * d, 0.0)
            is_cls = (ch >= 5) & (ch < NCH)
            t1 = (ch - 5) == lab
            valc = jnp.where(t1, p, 1.0 - p)
            cls = cls + jnp.where(is_cls, -_log_clamped(valc), 0.0)
        acc_v[pl.ds(4 * L, L)] = acc_v[pl.ds(4 * L, L)] + reg
        acc_v[pl.ds(5 * L, L)] = acc_v[pl.ds(5 * L, L)] + cls
        return 0
    lax.fori_loop(0, nwin, _winner, 0)

    pltpu.sync_copy(acc_v, part_hbm.at[wid])


@functools.partial(
    pl.kernel,
    out_type=jax.ShapeDtypeStruct((NW, 6 * L), jnp.float32),
    mesh=plsc.VectorSubcoreMesh(core_axis_name="c", subcore_axis_name="s"),
    compiler_params=pltpu.CompilerParams(needs_layout_passes=False),
    scratch_types=[
        pltpu.VMEM((6 * NGTS,), jnp.float32),
        pltpu.VMEM((NGTS,), jnp.int32),
        pltpu.VMEM((HBUF,), jnp.int32),
        pltpu.VMEM((HBUF,), jnp.int32),
        pltpu.VMEM((HBUF,), jnp.int32),
        pltpu.VMEM((HBUF,), jnp.float32),
        pltpu.VMEM((HBUF,), jnp.float32),
        pltpu.VMEM((HBUF,), jnp.float32),
        pltpu.VMEM((HBUF,), jnp.float32),
        pltpu.VMEM((HBUF,), jnp.float32),
        pltpu.VMEM((HBUF,), jnp.float32),
        pltpu.VMEM((NGTS + L,), jnp.int32),
        pltpu.VMEM((6 * L,), jnp.int32),
        pltpu.VMEM((6 * L,), jnp.float32),
        pltpu.VMEM((6 * L,), jnp.float32),
        pltpu.SemaphoreType.DMA,
        pltpu.SemaphoreType.DMA,
    ],
)
def _yolo_sc(out_hbm, gts_hbm, part_hbm, *rest):
    _yolo_sc_body(out_hbm, gts_hbm, part_hbm, *rest)


def kernel(out, gts):
    out_flat = out.reshape(-1)
    gts_t = gts.T.reshape(-1)
    parts = _yolo_sc(out_flat, gts_t)
    s = jnp.sum(parts.reshape(NW, 6, L), axis=(0, 2))
    msum = jnp.maximum(s[0], 1.0)
    nsum = jnp.maximum(s[1], 1.0)
    return s[4] / msum + s[2] / msum + 100.0 * s[3] / nsum + s[5] / (msum * NC)


# speedup vs baseline: 3.9927x; 3.9927x over previous
import functools

import jax
import jax.numpy as jnp
from jax import lax
from jax.experimental import pallas as pl
from jax.experimental.pallas import tpu as pltpu
from jax.experimental.pallas import tpu_sc as plsc

NB, NA, NG, NC = 16, 3, 52, 80
NGTS = 1920
NPOS = NG * NG
NCH = NC + 5
NW = 32
L = 16
GT_VECS = NGTS // L
PPT = 85
NRING = 5
GRID = PPT * 48

A_W = (0.28, 0.38, 0.9)
A_H = (0.22, 0.48, 0.78)
LN2 = 0.6931471805599453


def _ln(v):
    bits = lax.bitcast_convert_type(v, jnp.int32)
    e = ((bits >> 23) & 0xFF) - 127
    m = lax.bitcast_convert_type((bits & 0x007FFFFF) | 0x3F800000, jnp.float32)
    big = m > 1.4142135623730951
    m = jnp.where(big, m * 0.5, m)
    e = e + jnp.where(big, 1, 0)
    t = (m - 1.0) / (m + 1.0)
    t2 = t * t
    p = jnp.float32(2.0 / 9.0)
    p = 2.0 / 7.0 + t2 * p
    p = 2.0 / 5.0 + t2 * p
    p = 2.0 / 3.0 + t2 * p
    p = 2.0 + t2 * p
    return e.astype(jnp.float32) * LN2 + t * p


def _log_clamped(v):
    safe = jnp.maximum(v, jnp.float32(1e-37))
    return jnp.where(v < 1e-37, -100.0, jnp.maximum(_ln(safe), -100.0))


def _sigmoid(x):
    return 1.0 / (1.0 + jnp.exp(-x))


def _yolo_sc_body(out_hbm, gts_hbm, part_hbm,
                  gts_v, win_v, ign_v, blks, plist_v, acc_v, sems):
    cid = lax.axis_index("c")
    sid = lax.axis_index("s")
    wid = sid * 2 + cid
    iota = lax.iota(jnp.int32, L)
    fzero = jnp.zeros((L,), jnp.float32)

    plo = (NPOS * wid) // NW
    phi = (NPOS * (wid + 1)) // NW
    trip = phi - plo

    pltpu.sync_copy(gts_hbm, gts_v)
    for u in range(NRING):
        pos = plo + u
        j = pos // NG
        i = pos - j * NG
        pltpu.async_copy(out_hbm.at[j, i], blks[u], sems[u])

    def _init(q, _):
        sl = pl.ds(q * L, L)
        win_v[sl] = jnp.full((L,), -1, jnp.int32)
        ign_v[sl] = jnp.full((L,), 1.0, jnp.float32)
        return 0
    lax.fori_loop(0, GRID // L, _init, 0)

    for k in range(6):
        acc_v[pl.ds(k * L, L)] = fzero

    plo_v = jnp.full((L,), plo, jnp.int32)
    phi_v = jnp.full((L,), phi, jnp.int32)

    def _gt_pass(q, _):
        g = q * L + iota
        b = gts_v[pl.ds(q * L, L)].astype(jnp.int32)
        gx = gts_v[pl.ds(2 * NGTS + q * L, L)]
        gy = gts_v[pl.ds(3 * NGTS + q * L, L)]
        gw = gts_v[pl.ds(4 * NGTS + q * L, L)]
        gh = gts_v[pl.ds(5 * NGTS + q * L, L)]
        ious = []
        for a in range(NA):
            inter = jnp.minimum(gw, A_W[a]) * jnp.minimum(gh, A_H[a])
            union = gw * gh + A_W[a] * A_H[a] - inter
            ious.append(inter / union)
        best = jnp.zeros((L,), jnp.int32)
        bi = ious[0]
        best = jnp.where(ious[1] > bi, 1, best)
        bi = jnp.maximum(bi, ious[1])
        best = jnp.where(ious[2] > bi, 2, best)
        gi = (NG * gx).astype(jnp.int32)
        gj = (NG * gy).astype(jnp.int32)
        pos = gj * NG + gi
        inw = (pos >= plo_v) & (pos < phi_v)
        cbase = (pos - plo_v) * 48 + b
        lidx = jnp.clip(cbase + best * L, 0, GRID - 1)
        plsc.store_scatter(win_v, [lidx], g, mask=inw)
        for a in range(NA):
            ma = inw & (ious[a] > 0.5)
            la = jnp.clip(cbase + a * L, 0, GRID - 1)
            plsc.store_scatter(ign_v, [la], fzero, mask=ma)
        return 0
    lax.fori_loop(0, GT_VECS, _gt_pass, 0)

    chan_id = [jj * L + iota for jj in range(6)]

    def _pos_body(p_ofs, blk):
        validf = jnp.where(jnp.full((L,), p_ofs, jnp.int32) < trip, 1.0, 0.0)
        gbase = p_ofs * 48
        nwin = jnp.int32(0)
        for a in range(NA):
            win_a = win_v[pl.ds(gbase + a * L, L)]
            ign_a = ign_v[pl.ds(gbase + a * L, L)]
            x = plsc.load_gather(blk, [iota, jnp.full((L,), 4 + NCH * a,
                                                      jnp.int32)])
            isw = win_a >= 0
            objf = jnp.where(isw, validf, 0.0)
            noobjf = (validf - objf) * ign_a
            p = _sigmoid(x)
            val = jnp.where(isw, p, 1.0 - p)
            bce = -_log_clamped(val)
            acc_v[pl.ds(0, L)] = acc_v[pl.ds(0, L)] + objf
            acc_v[pl.ds(L, L)] = acc_v[pl.ds(L, L)] + noobjf
            acc_v[pl.ds(2 * L, L)] = acc_v[pl.ds(2 * L, L)] + objf * bce
            acc_v[pl.ds(3 * L, L)] = acc_v[pl.ds(3 * L, L)] + noobjf * bce
            winner = isw & (validf > 0.0)
            plsc.store_compressed(plist_v.at[pl.ds(nwin, L)], a * L + iota,
                                  mask=winner)
            nwin = nwin + jnp.sum(winner.astype(jnp.int32))

        def _winner(widx, _):
            lane = plsc.load_gather(plist_v, [jnp.full((L,), widx, jnp.int32)])
            a = lane // L
            b = lane - a * L
            g = plsc.load_gather(win_v, [gbase + lane])
            gxs = plsc.load_gather(gts_v, [2 * NGTS + g])
            gys = plsc.load_gather(gts_v, [3 * NGTS + g])
            gws = plsc.load_gather(gts_v, [4 * NGTS + g])
            ghs = plsc.load_gather(gts_v, [5 * NGTS + g])
            lab = plsc.load_gather(gts_v, [NGTS + g]).astype(jnp.int32)
            aw = jnp.where(a == 0, A_W[0], jnp.where(a == 1, A_W[1], A_W[2]))
            ah = jnp.where(a == 0, A_H[0], jnp.where(a == 1, A_H[1], A_H[2]))
            txf = NG * gxs
            tyf = NG * gys
            txs = txf - txf.astype(jnp.int32).astype(jnp.float32)
            tys = tyf - tyf.astype(jnp.int32).astype(jnp.float32)
            tw = _ln(gws / aw)
            th = _ln(ghs / ah)
            reg = fzero
            cls = fzero
            for jj in range(6):
                ch = chan_id[jj]
                col = a * NCH + jnp.minimum(ch, NCH - 1)
                xch = plsc.load_gather(blk, [b, col])
                pch = _sigmoid(xch)
                tgt = jnp.where(ch == 0, txs, jnp.where(ch == 1, tys,
                      jnp.where(ch == 2, tw, th)))
                vv = jnp.where(ch < 2, pch, xch)
                d = vv - tgt
                reg = reg + jnp.where(ch < 4, d * d, 0.0)
                is_cls = (ch >= 5) & (ch < NCH)
                t1 = (ch - 5) == lab
                valc = jnp.where(t1, pch, 1.0 - pch)
                cls = cls + jnp.where(is_cls, -_log_clamped(valc), 0.0)
            acc_v[pl.ds(4 * L, L)] = acc_v[pl.ds(4 * L, L)] + reg
            acc_v[pl.ds(5 * L, L)] = acc_v[pl.ds(5 * L, L)] + cls
            return 0
        lax.fori_loop(0, nwin, _winner, 0)

    def _ring(k, _):
        for u in range(NRING):
            p_ofs = k * NRING + u
            pltpu.make_async_copy(out_hbm.at[0, 0], blks[u], sems[u]).wait()
            _pos_body(p_ofs, blks[u])
            nxt = p_ofs + NRING

            @pl.when(nxt < PPT)
            def _():
                pos = plo + nxt
                j = pos // NG
                i = pos - j * NG
                pltpu.async_copy(out_hbm.at[j, i], blks[u], sems[u])
        return 0
    lax.fori_loop(0, PPT // NRING, _ring, 0)

    pltpu.sync_copy(acc_v, part_hbm.at[wid])


@functools.partial(
    pl.kernel,
    out_type=jax.ShapeDtypeStruct((NW, 6 * L), jnp.float32),
    mesh=plsc.VectorSubcoreMesh(core_axis_name="c", subcore_axis_name="s"),
    compiler_params=pltpu.CompilerParams(needs_layout_passes=False),
    scratch_types=(
        [pltpu.VMEM((6 * NGTS,), jnp.float32)]
        + [pltpu.VMEM((GRID,), jnp.int32)]
        + [pltpu.VMEM((GRID,), jnp.float32)]
        + [pltpu.VMEM((NB, NA * NCH), jnp.float32) for _ in range(NRING)]
        + [pltpu.VMEM((64,), jnp.int32)]
        + [pltpu.VMEM((6 * L,), jnp.float32)]
        + [pltpu.SemaphoreType.DMA for _ in range(NRING)]
    ),
)
def _yolo_sc(out_hbm, gts_hbm, part_hbm, *rest):
    _yolo_sc_body(out_hbm, gts_hbm, part_hbm,
                  rest[0], rest[1], rest[2], list(rest[3:3 + NRING]),
                  rest[3 + NRING], rest[4 + NRING],
                  list(rest[5 + NRING:5 + 2 * NRING]))


def kernel(out, gts):
    out_t = out.transpose(2, 3, 0, 1)
    gts_t = gts.T.reshape(-1)
    parts = _yolo_sc(out_t, gts_t)
    s = jnp.sum(parts.reshape(NW, 6, L), axis=(0, 2))
    msum = jnp.maximum(s[0], 1.0)
    nsum = jnp.maximum(s[1], 1.0)
    return s[4] / msum + s[2] / msum + 100.0 * s[3] / nsum + s[5] / (msum * NC)
